# baseline (speedup 1.0000x reference)
import functools

import jax
import jax.numpy as jnp
from jax.experimental import pallas as pl
from jax.experimental.pallas import tpu as pltpu
from jax.experimental.pallas import tpu_sc as plsc

F = 26
V = 100000
E = 32
B = 4096
NI = B * F
D_IN = F * E
H1, H2 = 512, 256
WIN = 128
BB = 512
NB = B // BB


def _sc_gather(emb2d, bias2d, flat_idx):
    mesh = plsc.VectorSubcoreMesh(core_axis_name="core", subcore_axis_name="subcore")

    @functools.partial(
        pl.kernel,
        out_type=(
            jax.ShapeDtypeStruct((NI, E), jnp.float32),
            jax.ShapeDtypeStruct((NI,), jnp.float32),
        ),
        mesh=mesh,
        compiler_params=pltpu.CompilerParams(use_tc_tiling_on_sc=False),
    )
    def k(emb_hbm, bias_hbm, i_hbm, emb_out, bias_out):
        def body(i_vmem, e_vmem, b_vmem):
            pltpu.sync_copy(emb_hbm.at[i_vmem.at[0]], e_vmem)
            pltpu.sync_copy(bias_hbm.at[i_vmem.at[0]], b_vmem)

        pltpu.emit_pipeline(
            body,
            grid=(NI // WIN,),
            in_specs=[pl.BlockSpec((1, WIN), index_map=lambda i: (0, i))],
            out_specs=[
                pl.BlockSpec((WIN, E), index_map=lambda i: (i, 0)),
                pl.BlockSpec((WIN,), index_map=lambda i: (i,)),
            ],
            core_axis_name=("core", "subcore"),
            dimension_semantics=(pltpu.PARALLEL,),
        )(i_hbm, emb_out, bias_out)

    return k(emb2d, bias2d, flat_idx)


def _mlp_body(x_ref, bv_ref, w1_ref, b1_ref, w2_ref, b2_ref, w3_ref, b3_ref, o_ref):
    x = x_ref[...]
    h = jnp.dot(x, w1_ref[...], preferred_element_type=jnp.float32) + b1_ref[...]
    h = jnp.maximum(h, 0.0)
    h = jnp.dot(h, w2_ref[...], preferred_element_type=jnp.float32) + b2_ref[...]
    h = jnp.maximum(h, 0.0)
    logits = jnp.sum(h * w3_ref[...], axis=1)
    logits = logits + b3_ref[0, 0] + jnp.sum(bv_ref[...], axis=1)
    o_ref[0, 0, :] = jax.nn.sigmoid(logits)


def _mlp(x, bias_bf, W1, b1, W2, b2, W3row, b3):
    return pl.pallas_call(
        _mlp_body,
        grid=(NB,),
        in_specs=[
            pl.BlockSpec((BB, D_IN), lambda i: (i, 0)),
            pl.BlockSpec((BB, F), lambda i: (i, 0)),
            pl.BlockSpec((D_IN, H1), lambda i: (0, 0)),
            pl.BlockSpec((1, H1), lambda i: (0, 0)),
            pl.BlockSpec((H1, H2), lambda i: (0, 0)),
            pl.BlockSpec((1, H2), lambda i: (0, 0)),
            pl.BlockSpec((1, H2), lambda i: (0, 0)),
            pl.BlockSpec((1, 1), lambda i: (0, 0)),
        ],
        out_specs=pl.BlockSpec((1, 1, BB), lambda i: (i, 0, 0)),
        out_shape=jax.ShapeDtypeStruct((NB, 1, BB), jnp.float32),
    )(x, bias_bf, W1, b1, W2, b2, W3row, b3)


def kernel(inputs, emb_table, bias_table, W1, b1, W2, b2, W3, b3):
    flat_idx = (inputs + jnp.arange(F, dtype=jnp.int32) * V).reshape(1, NI)
    emb2d = emb_table.reshape(F * V, E)
    bias1d = bias_table.reshape(F * V)
    x = emb2d[flat_idx.reshape(-1)].reshape(B, D_IN)
    bias_bf = bias1d[flat_idx.reshape(-1)].reshape(B, F)
    out = _mlp(
        x, bias_bf, W1, b1.reshape(1, H1), W2, b2.reshape(1, H2),
        W3.reshape(1, H2), b3.reshape(1, 1),
    )
    return out.reshape(B)


# speedup vs baseline: 9.3443x; 9.3443x over previous
import functools

import jax
import jax.numpy as jnp
from jax import lax
from jax.experimental import pallas as pl
from jax.experimental.pallas import tpu as pltpu
from jax.experimental.pallas import tpu_sc as plsc

F = 26
V = 100000
E = 32
B = 4096
NI = B * F
D_IN = F * E
H1, H2 = 512, 256
WIN = 128
BB = 512
NB = B // BB


NW = 32
NCH = NI // NW // WIN


def _sc_gather(emb2d, bias1d, idx2d):
    mesh = plsc.VectorSubcoreMesh(core_axis_name="core", subcore_axis_name="subcore")

    @functools.partial(
        pl.kernel,
        out_type=(
            jax.ShapeDtypeStruct((NW * NCH, WIN, E), jnp.float32),
            jax.ShapeDtypeStruct((NW * NCH, WIN), jnp.float32),
        ),
        mesh=mesh,
        compiler_params=pltpu.CompilerParams(use_tc_tiling_on_sc=False),
        scratch_types=[
            pltpu.VMEM((NCH, WIN), jnp.int32),
            pltpu.VMEM((NCH, WIN, E), jnp.float32),
            pltpu.VMEM((NCH, WIN), jnp.float32),
            pltpu.SemaphoreType.DMA,
            pltpu.SemaphoreType.DMA,
        ],
    )
    def k(emb_hbm, bias_hbm, idx_hbm, emb_out, bias_out,
          idx_v, rows_v, bias_v, sem_e, sem_b):
        wid = lax.axis_index("subcore") * 2 + lax.axis_index("core")
        base = wid * NCH
        pltpu.sync_copy(idx_hbm.at[pl.ds(base, NCH)], idx_v)

        @pl.loop(0, NCH)
        def _(j):
            pltpu.async_copy(emb_hbm.at[idx_v.at[j]], rows_v.at[j], sem_e)
            pltpu.async_copy(bias_hbm.at[idx_v.at[j]], bias_v.at[j], sem_b)

        pltpu.make_async_copy(emb_out.at[pl.ds(base, NCH)], rows_v, sem_e).wait()
        pltpu.make_async_copy(bias_out.at[pl.ds(base, NCH)], bias_v, sem_b).wait()
        pltpu.sync_copy(rows_v, emb_out.at[pl.ds(base, NCH)])
        pltpu.sync_copy(bias_v, bias_out.at[pl.ds(base, NCH)])

    return k(emb2d, bias1d, idx2d)


def _mlp_body(x_ref, bv_ref, w1_ref, b1_ref, w2_ref, b2_ref, w3_ref, b3_ref, o_ref):
    x = x_ref[...]
    h = jnp.dot(x, w1_ref[...], preferred_element_type=jnp.float32) + b1_ref[...]
    h = jnp.maximum(h, 0.0)
    h = jnp.dot(h, w2_ref[...], preferred_element_type=jnp.float32) + b2_ref[...]
    h = jnp.maximum(h, 0.0)
    logits = jnp.sum(h * w3_ref[...], axis=1)
    logits = logits + b3_ref[0, 0] + jnp.sum(bv_ref[...], axis=1)
    o_ref[0, 0, :] = jax.nn.sigmoid(logits)


def _mlp(x, bias_bf, W1, b1, W2, b2, W3row, b3):
    return pl.pallas_call(
        _mlp_body,
        grid=(NB,),
        in_specs=[
            pl.BlockSpec((BB, D_IN), lambda i: (i, 0)),
            pl.BlockSpec((BB, F), lambda i: (i, 0)),
            pl.BlockSpec((D_IN, H1), lambda i: (0, 0)),
            pl.BlockSpec((1, H1), lambda i: (0, 0)),
            pl.BlockSpec((H1, H2), lambda i: (0, 0)),
            pl.BlockSpec((1, H2), lambda i: (0, 0)),
            pl.BlockSpec((1, H2), lambda i: (0, 0)),
            pl.BlockSpec((1, 1), lambda i: (0, 0)),
        ],
        out_specs=pl.BlockSpec((1, 1, BB), lambda i: (i, 0, 0)),
        out_shape=jax.ShapeDtypeStruct((NB, 1, BB), jnp.float32),
    )(x, bias_bf, W1, b1, W2, b2, W3row, b3)


def kernel(inputs, emb_table, bias_table, W1, b1, W2, b2, W3, b3):
    flat_idx = (inputs + jnp.arange(F, dtype=jnp.int32) * V).reshape(NW * NCH, WIN)
    emb2d = emb_table.reshape(F * V, E)
    bias1d = bias_table.reshape(F * V)
    emb_rows, bias_rows = _sc_gather(emb2d, bias1d, flat_idx)
    x = emb_rows.reshape(B, D_IN)
    bias_bf = bias_rows.reshape(B, F)
    out = _mlp(
        x, bias_bf, W1, b1.reshape(1, H1), W2, b2.reshape(1, H2),
        W3.reshape(1, H2), b3.reshape(1, 1),
    )
    return out.reshape(B)
